# initial kernel scaffold (unmeasured)
import jax
import jax.numpy as jnp
from jax import lax
from jax.experimental import pallas as pl
from jax.experimental.pallas import tpu as pltpu

N_DEV = 8
N_TOK = 2048
D_IN = 512
D_OUT = 1024
N_EXP = 64
E_LOCAL = N_EXP // N_DEV
CHUNK = N_TOK // N_DEV
N_STEPS = 2 * (N_DEV - 1)


def kernel(x, router_W, route_idx, expert_W):
    def body(x_ref, rw_ref, idx_ref, ew_ref, out_ref,
             comm_ref, send_sems, recv_sems, credit_sem):
        d = lax.axis_index("i")
        left = lax.rem(d - 1 + N_DEV, N_DEV)
        right = lax.rem(d + 1, N_DEV)

        xv = x_ref[:, :]
        scores = jnp.dot(xv, rw_ref[:, :], preferred_element_type=jnp.float32)
        mx = jnp.max(scores, axis=-1, keepdims=True)
        ex = jnp.exp(scores - mx)
        probs = ex / jnp.sum(ex, axis=-1, keepdims=True)
        idx0 = idx_ref[:, 0:1]
        idx1 = idx_ref[:, 1:2]
        cols = lax.broadcasted_iota(jnp.int32, (N_TOK, N_EXP), 1)
        g0 = jnp.sum(jnp.where(cols == idx0, probs, 0.0), axis=1, keepdims=True)
        g1 = jnp.sum(jnp.where(cols == idx1, probs, 0.0), axis=1, keepdims=True)
        gsum = g0 + g1
        g0 = g0 / gsum
        g1 = g1 / gsum
        for e_i in range(E_LOCAL):
            ge = d * E_LOCAL + e_i
            w = jnp.where(idx0 == ge, g0, 0.0) + jnp.where(idx1 == ge, g1, 0.0)
            contrib = jnp.dot(xv * w, ew_ref[e_i],
                              preferred_element_type=jnp.float32)
            if e_i == 0:
                out_ref[:, :] = contrib
            else:
                out_ref[:, :] = out_ref[:, :] + contrib

        bsem = pltpu.get_barrier_semaphore()
        for nbr in (left, right):
            pl.semaphore_signal(bsem, inc=1, device_id=(nbr,),
                                device_id_type=pl.DeviceIdType.MESH)
        pl.semaphore_wait(bsem, 2)

        for t in range(N_STEPS):
            slot = t % 2
            if t < N_DEV - 1:
                send_c = lax.rem(d - t + 2 * N_DEV, N_DEV)
                recv_c = lax.rem(d - t - 1 + 2 * N_DEV, N_DEV)
            else:
                u = t - (N_DEV - 1)
                send_c = lax.rem(d + 1 - u + 2 * N_DEV, N_DEV)
                recv_c = lax.rem(d - u + 2 * N_DEV, N_DEV)

            if t >= 2:
                pl.semaphore_wait(credit_sem, 1)

            rdma = pltpu.make_async_remote_copy(
                src_ref=out_ref.at[pl.ds(send_c * CHUNK, CHUNK), :],
                dst_ref=comm_ref.at[slot],
                send_sem=send_sems.at[slot],
                recv_sem=recv_sems.at[slot],
                device_id=(right,),
                device_id_type=pl.DeviceIdType.MESH,
            )
            rdma.start()
            rdma.wait()

            if t < N_DEV - 1:
                out_ref[pl.ds(recv_c * CHUNK, CHUNK), :] = (
                    out_ref[pl.ds(recv_c * CHUNK, CHUNK), :] + comm_ref[slot]
                )
            else:
                out_ref[pl.ds(recv_c * CHUNK, CHUNK), :] = comm_ref[slot]

            if t < N_STEPS - 2:
                pl.semaphore_signal(credit_sem, inc=1, device_id=(left,),
                                    device_id_type=pl.DeviceIdType.MESH)

    return pl.pallas_call(
        body,
        out_shape=jax.ShapeDtypeStruct((N_TOK, D_OUT), jnp.float32),
        in_specs=[
            pl.BlockSpec(memory_space=pltpu.VMEM),
            pl.BlockSpec(memory_space=pltpu.VMEM),
            pl.BlockSpec(memory_space=pltpu.VMEM),
            pl.BlockSpec(memory_space=pltpu.VMEM),
        ],
        out_specs=pl.BlockSpec(memory_space=pltpu.VMEM),
        scratch_shapes=[
            pltpu.VMEM((2, CHUNK, D_OUT), jnp.float32),
            pltpu.SemaphoreType.DMA((2,)),
            pltpu.SemaphoreType.DMA((2,)),
            pltpu.SemaphoreType.REGULAR,
        ],
        compiler_params=pltpu.CompilerParams(collective_id=0),
    )(x, router_W, route_idx, expert_W)


# baseline (device time: 237689 ns/iter reference)
import jax
import jax.numpy as jnp
from jax import lax
from jax.experimental import pallas as pl
from jax.experimental.pallas import tpu as pltpu

N_DEV = 8
N_TOK = 2048
D_IN = 512
D_OUT = 1024
N_EXP = 64
E_LOCAL = N_EXP // N_DEV
CHUNK = N_TOK // N_DEV
N_STEPS = 2 * (N_DEV - 1)


def kernel(x, router_W, route_idx, expert_W):
    def body(x_ref, rw_ref, idx_ref, ew_ref, out_ref,
             comm_ref, send_sems, recv_sems, credit_sem):
        d = lax.axis_index("i")
        left = lax.rem(d - 1 + N_DEV, N_DEV)
        right = lax.rem(d + 1, N_DEV)

        xv = x_ref[:, :]
        scores = jnp.dot(xv, rw_ref[:, :], preferred_element_type=jnp.float32)
        mx = jnp.max(scores, axis=-1, keepdims=True)
        ex = jnp.exp(scores - mx)
        probs = ex / jnp.sum(ex, axis=-1, keepdims=True)
        idx0 = idx_ref[:, 0:1]
        idx1 = idx_ref[:, 1:2]
        cols = lax.broadcasted_iota(jnp.int32, (N_TOK, N_EXP), 1)
        g0 = jnp.sum(jnp.where(cols == idx0, probs, 0.0), axis=1, keepdims=True)
        g1 = jnp.sum(jnp.where(cols == idx1, probs, 0.0), axis=1, keepdims=True)
        gsum = g0 + g1
        g0 = g0 / gsum
        g1 = g1 / gsum
        for e_i in range(E_LOCAL):
            ge = d * E_LOCAL + e_i
            w = jnp.where(idx0 == ge, g0, 0.0) + jnp.where(idx1 == ge, g1, 0.0)
            contrib = jnp.dot(xv * w, ew_ref[e_i],
                              preferred_element_type=jnp.float32)
            if e_i == 0:
                out_ref[:, :] = contrib
            else:
                out_ref[:, :] = out_ref[:, :] + contrib

        bsem = pltpu.get_barrier_semaphore()
        for nbr in (left, right):
            pl.semaphore_signal(bsem, inc=1, device_id=(nbr,),
                                device_id_type=pl.DeviceIdType.MESH)
        pl.semaphore_wait(bsem, 2)

        for t in range(N_STEPS):
            slot = t % 2
            if t < N_DEV - 1:
                send_c = lax.rem(d - t + 2 * N_DEV, N_DEV)
                recv_c = lax.rem(d - t - 1 + 2 * N_DEV, N_DEV)
            else:
                u = t - (N_DEV - 1)
                send_c = lax.rem(d + 1 - u + 2 * N_DEV, N_DEV)
                recv_c = lax.rem(d - u + 2 * N_DEV, N_DEV)

            if t >= 2:
                pl.semaphore_wait(credit_sem, 1)

            rdma = pltpu.make_async_remote_copy(
                src_ref=out_ref.at[pl.ds(send_c * CHUNK, CHUNK), :],
                dst_ref=comm_ref.at[slot],
                send_sem=send_sems.at[slot],
                recv_sem=recv_sems.at[slot],
                device_id=(right,),
                device_id_type=pl.DeviceIdType.MESH,
            )
            rdma.start()
            rdma.wait()

            if t < N_DEV - 1:
                out_ref[pl.ds(recv_c * CHUNK, CHUNK), :] = (
                    out_ref[pl.ds(recv_c * CHUNK, CHUNK), :] + comm_ref[slot]
                )
            else:
                out_ref[pl.ds(recv_c * CHUNK, CHUNK), :] = comm_ref[slot]

            if t < N_STEPS - 2:
                pl.semaphore_signal(credit_sem, inc=1, device_id=(left,),
                                    device_id_type=pl.DeviceIdType.MESH)

    return pl.pallas_call(
        body,
        out_shape=jax.ShapeDtypeStruct((N_TOK, D_OUT), jnp.float32),
        in_specs=[
            pl.BlockSpec(memory_space=pltpu.VMEM),
            pl.BlockSpec(memory_space=pltpu.VMEM),
            pl.BlockSpec(memory_space=pltpu.VMEM),
            pl.BlockSpec(memory_space=pltpu.VMEM),
        ],
        out_specs=pl.BlockSpec(memory_space=pltpu.VMEM),
        scratch_shapes=[
            pltpu.VMEM((2, CHUNK, D_OUT), jnp.float32),
            pltpu.SemaphoreType.DMA((2,)),
            pltpu.SemaphoreType.DMA((2,)),
            pltpu.SemaphoreType.REGULAR,
        ],
        compiler_params=pltpu.CompilerParams(
            collective_id=0,
            vmem_limit_bytes=100 * 1024 * 1024,
        ),
    )(x, router_W, route_idx, expert_W)


# device time: 81064 ns/iter; 2.9321x vs baseline; 2.9321x over previous
import jax
import jax.numpy as jnp
from jax import lax
from jax.experimental import pallas as pl
from jax.experimental.pallas import tpu as pltpu

N_DEV = 8
N_TOK = 2048
D_IN = 512
D_OUT = 1024
N_EXP = 64
E_LOCAL = N_EXP // N_DEV

PARTS = [
    (0, 704, (1, 3, 4)),
    (704, 704, (3, 4, 1)),
    (1408, 640, (4, 1, 3)),
]
COMM_OFF = [0, 616, 1232]
COMM_ROWS = 1792


def _bit(dv, mask):
    if mask == 1:
        return (dv ^ (dv >> 1)) & 1
    if mask == 3:
        return (dv >> 1) & 1
    return (dv >> 2) & 1


def kernel(x, router_W, route_idx, expert_W):
    def body(x_ref, rw_ref, idx_ref, ew_ref, out_ref,
             wscr, work, comm_ref, send_sems, recv_sems):
        d = lax.axis_index("i")

        xv = x_ref[:, :]
        scores = jnp.dot(xv, rw_ref[:, :], preferred_element_type=jnp.float32)
        mx = jnp.max(scores, axis=-1, keepdims=True)
        ex = jnp.exp(scores - mx)
        probs = ex / jnp.sum(ex, axis=-1, keepdims=True)
        idx0 = idx_ref[:, 0:1]
        idx1 = idx_ref[:, 1:2]
        cols = lax.broadcasted_iota(jnp.int32, (N_TOK, N_EXP), 1)
        g0 = jnp.sum(jnp.where(cols == idx0, probs, 0.0), axis=1, keepdims=True)
        g1 = jnp.sum(jnp.where(cols == idx1, probs, 0.0), axis=1, keepdims=True)
        gsum = g0 + g1
        g0 = g0 / gsum
        g1 = g1 / gsum
        for e_i in range(E_LOCAL):
            ge = d * E_LOCAL + e_i
            wscr[:, e_i:e_i + 1] = (
                jnp.where(idx0 == ge, g0, 0.0) + jnp.where(idx1 == ge, g1, 0.0)
            )

        def compute_rows(row_start, n_rows):
            xs = x_ref[pl.ds(row_start, n_rows), :]
            ws = wscr[pl.ds(row_start, n_rows), :]
            acc = jnp.dot(xs * ws[:, 0:1], ew_ref[0],
                          preferred_element_type=jnp.float32)
            for e_i in range(1, E_LOCAL):
                acc = acc + jnp.dot(xs * ws[:, e_i:e_i + 1], ew_ref[e_i],
                                    preferred_element_type=jnp.float32)
            work[pl.ds(row_start, n_rows), :] = acc.astype(jnp.bfloat16)

        H = [R // 2 for _, R, _ in PARTS]
        Q = [R // 4 for _, R, _ in PARTS]
        E = [R // 8 for _, R, _ in PARTS]
        b = [[_bit(d, masks[s]) for s in range(3)] for _, _, masks in PARTS]
        off0 = [r0 + b[p][0] * H[p] for p, (r0, _, _) in enumerate(PARTS)]
        off1 = [off0[p] + b[p][1] * Q[p] for p in range(3)]
        off2 = [off1[p] + b[p][2] * E[p] for p in range(3)]
        qs_off = [off0[p] + (1 - b[p][1]) * Q[p] for p in range(3)]
        es_off = [off1[p] + (1 - b[p][2]) * E[p] for p in range(3)]
        creg0 = [COMM_OFF[p] for p in range(3)]
        creg1 = [COMM_OFF[p] + H[p] for p in range(3)]
        creg2 = [COMM_OFF[p] + H[p] + Q[p] for p in range(3)]

        for p in range(3):
            compute_rows(PARTS[p][0] + (1 - b[p][0]) * H[p], H[p])

        bsem = pltpu.get_barrier_semaphore()
        for m in (1, 3, 4):
            pl.semaphore_signal(bsem, inc=1, device_id=(d ^ m,),
                                device_id_type=pl.DeviceIdType.MESH)
        pl.semaphore_wait(bsem, 3)

        def rs_rdma(p, s, src_off, n, creg):
            r = pltpu.make_async_remote_copy(
                src_ref=work.at[pl.ds(src_off, n), :],
                dst_ref=comm_ref.at[pl.ds(creg, n), :],
                send_sem=send_sems.at[p, s],
                recv_sem=recv_sems.at[p, s],
                device_id=(d ^ PARTS[p][2][s],),
                device_id_type=pl.DeviceIdType.MESH,
            )
            r.start()
            return r

        def add_from_comm(dst_off, creg_off, n):
            work[pl.ds(dst_off, n), :] = (
                work[pl.ds(dst_off, n), :].astype(jnp.float32)
                + comm_ref[pl.ds(creg_off, n), :].astype(jnp.float32)
            ).astype(jnp.bfloat16)

        rs0 = [rs_rdma(p, 0, PARTS[p][0] + (1 - b[p][0]) * H[p], H[p],
                       creg0[p]) for p in range(3)]
        for p in range(3):
            compute_rows(qs_off[p], Q[p])
        for p in range(3):
            rs0[p].wait()
            add_from_comm(qs_off[p], creg0[p] + (qs_off[p] - off0[p]), Q[p])

        rs1 = [rs_rdma(p, 1, qs_off[p], Q[p], creg1[p]) for p in range(3)]
        for p in range(3):
            compute_rows(off1[p], Q[p])
        for p in range(3):
            add_from_comm(off1[p], creg0[p] + (off1[p] - off0[p]), Q[p])
        for p in range(3):
            rs1[p].wait()
            add_from_comm(es_off[p], creg1[p] + (es_off[p] - off1[p]), E[p])

        rs2 = [rs_rdma(p, 2, es_off[p], E[p], creg2[p]) for p in range(3)]
        for p in range(3):
            add_from_comm(off2[p], creg1[p] + (off2[p] - off1[p]), E[p])
        for p in range(3):
            rs2[p].wait()
            add_from_comm(off2[p], creg2[p], E[p])

        off = list(off2)
        size = list(E)

        for s in range(3):
            started = []
            for p_i, (r0, R, masks) in enumerate(PARTS):
                m = masks[2 - s]
                b = _bit(d, m)
                sz = size[p_i]
                rdma = pltpu.make_async_remote_copy(
                    src_ref=work.at[pl.ds(off[p_i], sz), :],
                    dst_ref=work.at[pl.ds(off[p_i], sz), :],
                    send_sem=send_sems.at[p_i, 3 + s],
                    recv_sem=recv_sems.at[p_i, 3 + s],
                    device_id=(d ^ m,),
                    device_id_type=pl.DeviceIdType.MESH,
                )
                rdma.start()
                off[p_i] = off[p_i] - b * sz
                size[p_i] = 2 * sz
                started.append(rdma)
            for rdma in started:
                rdma.wait()
        out_ref[:, :] = work[:, :].astype(jnp.float32)

    return pl.pallas_call(
        body,
        out_shape=jax.ShapeDtypeStruct((N_TOK, D_OUT), jnp.float32),
        in_specs=[
            pl.BlockSpec(memory_space=pltpu.VMEM),
            pl.BlockSpec(memory_space=pltpu.VMEM),
            pl.BlockSpec(memory_space=pltpu.VMEM),
            pl.BlockSpec(memory_space=pltpu.VMEM),
        ],
        out_specs=pl.BlockSpec(memory_space=pltpu.VMEM),
        scratch_shapes=[
            pltpu.VMEM((N_TOK, E_LOCAL), jnp.float32),
            pltpu.VMEM((N_TOK, D_OUT), jnp.bfloat16),
            pltpu.VMEM((COMM_ROWS, D_OUT), jnp.bfloat16),
            pltpu.SemaphoreType.DMA((3, 6)),
            pltpu.SemaphoreType.DMA((3, 6)),
        ],
        compiler_params=pltpu.CompilerParams(
            collective_id=0,
            vmem_limit_bytes=100 * 1024 * 1024,
        ),
    )(x, router_W, route_idx, expert_W)
